# Initial kernel scaffold; baseline (speedup 1.0000x reference)
import jax
import jax.numpy as jnp
from jax.experimental import pallas as pl


def kernel(x, edge_index, W1_l, b1, W1_r, W2_l, b2, W2_r):
    raise NotImplementedError("write your pallas kernel here")


# speedup vs baseline: 3.2699x; 3.2699x over previous
import functools

import jax
import jax.numpy as jnp
from jax import lax
from jax.experimental import pallas as pl
from jax.experimental.pallas import tpu as pltpu
from jax.experimental.pallas import tpu_sc as plsc

N = 10000
D = 128
E = 320000

NC = 2
NS = 16
NW = NC * NS

RPT = 640
NPAD = NS * RPT

KCH = 2
EROWS = 2560
EPAD = EROWS * 128
RW = EROWS // NW
CW = EROWS // NS

_mesh = plsc.VectorSubcoreMesh(core_axis_name="c", subcore_axis_name="s")


def _agg_body(first, x_hbm, src_hbm, dst_hbm, invc_hbm, z2_hbm, z1_hbm,
              ones_hbm, part_out, invc_out, acc_sh, cnt_sh, srcv, dstv,
              rows_v, cnt_v, ones_v, sem):
    cid = lax.axis_index("c")
    sid = lax.axis_index("s")
    wid = cid * NS + sid
    r0 = sid * RPT

    pltpu.sync_copy(z2_hbm.at[pl.ds(r0, RPT)], acc_sh.at[pl.ds(r0, RPT)])
    if first:
        pltpu.sync_copy(z1_hbm.at[pl.ds(r0, RPT)], cnt_sh.at[pl.ds(r0, RPT)])
        pltpu.sync_copy(ones_hbm, ones_v)
    plsc.subcore_barrier()

    if first:
        def count_body(g, carry):
            row = sid * CW + g * KCH
            pltpu.sync_copy(dst_hbm.at[pl.ds(row, KCH)], dstv)
            for j in range(KCH):
                pltpu.sync_copy(ones_v, cnt_sh.at[dstv.at[j]], add=True)
            return carry
        lax.fori_loop(0, CW // KCH, count_body, 0)
        plsc.subcore_barrier()
        pltpu.sync_copy(cnt_sh.at[pl.ds(r0, RPT)], cnt_v)
        for i in range(RPT // 16):
            v = cnt_v[pl.ds(i * 16, 16)]
            cnt_v[pl.ds(i * 16, 16)] = 1.0 / jnp.maximum(v, 1.0)
        pltpu.sync_copy(cnt_v, invc_out.at[cid, pl.ds(r0, RPT)])
    else:
        pltpu.sync_copy(invc_hbm.at[0, pl.ds(r0, RPT)], cnt_v)

    def edge_body(g, carry):
        row = wid * RW + g * KCH
        pltpu.sync_copy(src_hbm.at[pl.ds(row, KCH)], srcv)
        pltpu.sync_copy(dst_hbm.at[pl.ds(row, KCH)], dstv)
        descs = []
        for j in range(KCH):
            descs.append(pltpu.async_copy(
                x_hbm.at[srcv.at[j]], rows_v.at[pl.ds(j * 128, 128)], sem))
        for d in descs:
            d.wait()
        for j in range(KCH):
            pltpu.sync_copy(rows_v.at[pl.ds(j * 128, 128)],
                            acc_sh.at[dstv.at[j]], add=True)
        return carry
    lax.fori_loop(0, RW // KCH, edge_body, 0)
    plsc.subcore_barrier()

    for c5 in range(RPT // 128):
        rr = r0 + c5 * 128
        pltpu.sync_copy(acc_sh.at[pl.ds(rr, 128)], rows_v.at[pl.ds(0, 128)])

        def scale_body(rg, carry, c5=c5):
            invvec = cnt_v[pl.ds(c5 * 128 + rg * 16, 16)]
            for li in range(16):
                r = rg * 16 + li
                inv = invvec[li]
                for l in range(8):
                    sl = pl.ds(l * 16, 16)
                    rows_v[r, sl] = rows_v[r, sl] * inv
            return carry
        lax.fori_loop(0, 8, scale_body, 0)
        pltpu.sync_copy(rows_v.at[pl.ds(0, 128)],
                        part_out.at[cid, pl.ds(rr, 128)])


def _make_agg(first):
    out_type = [jax.ShapeDtypeStruct((NC, NPAD, D), jnp.float32),
                jax.ShapeDtypeStruct((NC, NPAD), jnp.float32)]
    return pl.kernel(
        functools.partial(_agg_body, first),
        out_type=out_type,
        mesh=_mesh,
        scratch_types=[
            pltpu.VMEM_SHARED((NPAD, D), jnp.float32),
            pltpu.VMEM_SHARED((NPAD,), jnp.float32),
            pltpu.VMEM((KCH, 128), jnp.int32),
            pltpu.VMEM((KCH, 128), jnp.int32),
            pltpu.VMEM((KCH * 128, D), jnp.float32),
            pltpu.VMEM((RPT,), jnp.float32),
            pltpu.VMEM((128,), jnp.float32),
            pltpu.SemaphoreType.DMA,
        ],
        name="sage_agg1" if first else "sage_agg2",
    )


_agg1 = _make_agg(True)
_agg2 = _make_agg(False)

_BLK = 400


def _dense_body(leaky, p_ref, x_ref, wl_ref, wr_ref, b_ref, o_ref):
    mean = p_ref[0] + p_ref[1]
    o = jnp.dot(mean, wl_ref[...], preferred_element_type=jnp.float32)
    o = o + jnp.dot(x_ref[...], wr_ref[...], preferred_element_type=jnp.float32)
    o = o + b_ref[...]
    if leaky:
        o = jnp.where(o >= 0.0, o, o * 0.01)
    o_ref[...] = o


def _dense(part, xin, wlT, wrT, b2d, leaky):
    grid = (N // _BLK,)
    return pl.pallas_call(
        functools.partial(_dense_body, leaky),
        grid=grid,
        in_specs=[
            pl.BlockSpec((NC, _BLK, D), lambda i: (0, i, 0)),
            pl.BlockSpec((_BLK, D), lambda i: (i, 0)),
            pl.BlockSpec((D, D), lambda i: (0, 0)),
            pl.BlockSpec((D, D), lambda i: (0, 0)),
            pl.BlockSpec((1, D), lambda i: (0, 0)),
        ],
        out_specs=pl.BlockSpec((_BLK, D), lambda i: (i, 0)),
        out_shape=jax.ShapeDtypeStruct((N, D), jnp.float32),
    )(part, xin, wlT, wrT, b2d)


def kernel(x, edge_index, W1_l, b1, W1_r, W2_l, b2, W2_r):
    src = edge_index[0].astype(jnp.int32)
    dst = edge_index[1].astype(jnp.int32)
    npad_e = EPAD - E
    src2d = jnp.concatenate(
        [src, jnp.zeros((npad_e,), jnp.int32)]).reshape(EROWS, 128)
    dst2d = jnp.concatenate(
        [dst, jnp.full((npad_e,), N, jnp.int32)]).reshape(EROWS, 128)
    z2 = jnp.zeros((NPAD, D), jnp.float32)
    z1 = jnp.zeros((NPAD,), jnp.float32)
    ones = jnp.ones((128,), jnp.float32)
    inv_dummy = jnp.zeros((NC, NPAD), jnp.float32)

    part1, invc = _agg1(x, src2d, dst2d, inv_dummy, z2, z1, ones)
    h = _dense(part1, x, W1_l.T, W1_r.T, b1.reshape(1, D), True)
    part2, _ = _agg2(h, src2d, dst2d, invc, z2, z1, ones)
    out = _dense(part2, h, W2_l.T, W2_r.T, b2.reshape(1, D), False)
    return out
